# baseline (speedup 1.0000x reference)
import functools

import jax
import jax.numpy as jnp
from jax import lax
from jax.experimental import pallas as pl
from jax.experimental.pallas import tpu as pltpu
from jax.experimental.pallas import tpu_sc as plsc

_N_USERS = 1000000
_N_ROWS = 1100000
_BATCH = 16384
_NF = 32
_L = 16

_info = plsc.get_sparse_core_info()
_NC, _NS = _info.num_cores, _info.num_subcores
_NW = _NC * _NS
_P = _BATCH // _NW
_CH = 128
_NCH = _P // _CH
_NG = _CH // _L

_mesh = plsc.VectorSubcoreMesh(core_axis_name="c", subcore_axis_name="s")


@functools.partial(
    pl.kernel,
    mesh=_mesh,
    out_type=jax.ShapeDtypeStruct((_BATCH,), jnp.float32),
    compiler_params=pltpu.CompilerParams(
        needs_layout_passes=False, use_tc_tiling_on_sc=False),
    scratch_types=[
        pltpu.VMEM((_NCH, _CH), jnp.int32),
        pltpu.VMEM((_NCH, _CH), jnp.int32),
        pltpu.VMEM((_NF, _CH), jnp.float32),
        pltpu.VMEM((_NF, _CH), jnp.float32),
        pltpu.VMEM((_NF, _CH), jnp.float32),
        pltpu.VMEM((_NF, _CH), jnp.float32),
        pltpu.VMEM((_P,), jnp.float32),
        pltpu.VMEM((_P,), jnp.float32),
        pltpu.VMEM((_P,), jnp.float32),
        pltpu.VMEM((_L,), jnp.float32),
        pltpu.SemaphoreType.DMA,
        pltpu.SemaphoreType.DMA,
        pltpu.SemaphoreType.DMA,
    ],
)
def _fm_sc(users_hbm, items_hbm, emb_hbm, fc_hbm, bias_hbm, out_hbm,
           idxu_v, idxi_v, eu0_v, eu1_v, ei0_v, ei1_v,
           fu_v, fi_v, out_v, bias_v, sem0, sem1, semf):
    wid = lax.axis_index("s") * _NC + lax.axis_index("c")
    base = wid * _P

    for c in range(_NCH):
        pltpu.sync_copy(users_hbm.at[pl.ds(base + c * _CH, _CH)], idxu_v.at[c])
        pltpu.sync_copy(items_hbm.at[pl.ds(base + c * _CH, _CH)], idxi_v.at[c])
    pltpu.sync_copy(bias_hbm, bias_v.at[pl.ds(0, 1)])

    for c in range(_NCH):
        for j in range(_CH // _L):
            sl = pl.ds(j * _L, _L)
            idxi_v[c, sl] = idxi_v[c, sl] + _N_USERS

    fcops = []
    for c in range(_NCH):
        dst = pl.ds(c * _CH, _CH)
        fcops.append(pltpu.async_copy(fc_hbm.at[idxu_v.at[c]], fu_v.at[dst], semf))
        fcops.append(pltpu.async_copy(fc_hbm.at[idxi_v.at[c]], fi_v.at[dst], semf))

    eu_slots = (eu0_v, eu1_v)
    ei_slots = (ei0_v, ei1_v)

    def fire(c):
        s = sem0 if c % 2 == 0 else sem1
        eu_v, ei_v = eu_slots[c % 2], ei_slots[c % 2]
        ops = []
        for k in range(_NF):
            col = emb_hbm.at[pl.ds(k * _N_ROWS, _N_ROWS)]
            ops.append(pltpu.async_copy(col.at[idxu_v.at[c]], eu_v.at[k], s))
            ops.append(pltpu.async_copy(col.at[idxi_v.at[c]], ei_v.at[k], s))
        return ops

    pend = {0: fire(0), 1: fire(1)}
    for cp in fcops:
        cp.wait()
    bias_s = bias_v[...][0]

    for c in range(_NCH):
        for cp in pend.pop(c):
            cp.wait()
        eu_v, ei_v = eu_slots[c % 2], ei_slots[c % 2]

        def body(g, carry, c=c, eu_v=eu_v, ei_v=ei_v):
            sl16 = pl.ds(c * _CH + g * _L, _L)
            gsl = pl.ds(g * _L, _L)
            acc = fu_v[sl16] + fi_v[sl16] + bias_s
            for k in range(_NF):
                acc = acc + eu_v[k, gsl] * ei_v[k, gsl]
            out_v[sl16] = acc
            return carry

        lax.fori_loop(0, _NG, body, 0)
        if c + 2 < _NCH:
            pend[c + 2] = fire(c + 2)

    pltpu.sync_copy(out_v, out_hbm.at[pl.ds(base, _P)])


def kernel(users_feat, items_feat, emb_w, fc_w, bias):
    emb_kmajor = emb_w.T.reshape(-1)
    return _fm_sc(users_feat, items_feat, emb_kmajor, fc_w.reshape(-1), bias)


# speedup vs baseline: 14.2327x; 14.2327x over previous
import functools

import jax
import jax.numpy as jnp
from jax import lax
from jax.experimental import pallas as pl
from jax.experimental.pallas import tpu as pltpu
from jax.experimental.pallas import tpu_sc as plsc

_N_USERS = 1000000
_N_ROWS = 1100000
_NMAIN = 1099904
_NTAIL = _N_ROWS - _NMAIN
_BATCH = 16384
_NF = 32
_L = 16

_info = plsc.get_sparse_core_info()
_NC, _NS = _info.num_cores, _info.num_subcores
_NW = _NC * _NS
_P = _BATCH // _NW
_CH = 128
_NCH = _P // _CH
_NG = _CH // _L

_mesh = plsc.VectorSubcoreMesh(core_axis_name="c", subcore_axis_name="s")


@functools.partial(
    pl.kernel,
    mesh=_mesh,
    out_type=jax.ShapeDtypeStruct((_BATCH,), jnp.float32),
    compiler_params=pltpu.CompilerParams(
        needs_layout_passes=False, use_tc_tiling_on_sc=False),
    scratch_types=[
        pltpu.VMEM((_NCH, _CH), jnp.int32),
        pltpu.VMEM((_NCH, _CH), jnp.int32),
        pltpu.VMEM((8, _CH), jnp.int32),
        pltpu.VMEM((8, _CH), jnp.int32),
        pltpu.VMEM((8, _CH), jnp.int32),
        pltpu.VMEM((8, _CH), jnp.int32),
        pltpu.VMEM((_NF, _CH), jnp.float32),
        pltpu.VMEM((_NF, _CH), jnp.float32),
        pltpu.VMEM((_NF, _CH), jnp.float32),
        pltpu.VMEM((_NF, _CH), jnp.float32),
        pltpu.VMEM((_P,), jnp.float32),
        pltpu.VMEM((_P,), jnp.float32),
        pltpu.VMEM((_P,), jnp.float32),
        pltpu.VMEM((_L,), jnp.float32),
        pltpu.VMEM((_NF * _NTAIL,), jnp.float32),
        pltpu.SemaphoreType.DMA,
        pltpu.SemaphoreType.DMA,
        pltpu.SemaphoreType.DMA,
    ],
)
def _fm_sc(users_hbm, items_hbm, t0_hbm, t1_hbm, t2_hbm, t3_hbm, tail_hbm,
           fc_hbm, bias_hbm, out_hbm,
           idxu_v, idxi_v, au0_v, au1_v, ai0_v, ai1_v,
           eu0_v, eu1_v, ei0_v, ei1_v,
           fu_v, fi_v, out_v, bias_v, tail_v, sem0, sem1, semf):
    wid = lax.axis_index("s") * _NC + lax.axis_index("c")
    base = wid * _P

    for c in range(_NCH):
        pltpu.sync_copy(users_hbm.at[pl.ds(base + c * _CH, _CH)], idxu_v.at[c])
        pltpu.sync_copy(items_hbm.at[pl.ds(base + c * _CH, _CH)], idxi_v.at[c])
    pltpu.sync_copy(bias_hbm, bias_v.at[pl.ds(0, 1)])
    pltpu.sync_copy(tail_hbm, tail_v)

    for c in range(_NCH):
        for j in range(_CH // _L):
            sl = pl.ds(j * _L, _L)
            idxi_v[c, sl] = idxi_v[c, sl] + _N_USERS

    fcops = []
    for c in range(_NCH):
        dst = pl.ds(c * _CH, _CH)
        fcops.append(pltpu.async_copy(fc_hbm.at[idxu_v.at[c]], fu_v.at[dst], semf))
        fcops.append(pltpu.async_copy(fc_hbm.at[idxi_v.at[c]], fi_v.at[dst], semf))

    tabs = (t0_hbm, t1_hbm, t2_hbm, t3_hbm)
    au_slots = (au0_v, au1_v)
    ai_slots = (ai0_v, ai1_v)
    eu_slots = (eu0_v, eu1_v)
    ei_slots = (ei0_v, ei1_v)

    def fire(c):
        s = sem0 if c % 2 == 0 else sem1
        slot = c % 2
        au_v, ai_v = au_slots[slot], ai_slots[slot]
        eu_v, ei_v = eu_slots[slot], ei_slots[slot]
        for j in range(_CH // _L):
            sl = pl.ds(j * _L, _L)
            tu = idxu_v[c, sl]
            bu = lax.shift_left(lax.shift_right_logical(tu, 7), 10) + (tu & 127)
            ti = jnp.minimum(idxi_v[c, sl], _NMAIN - 1)
            bi = lax.shift_left(lax.shift_right_logical(ti, 7), 10) + (ti & 127)
            for ki in range(8):
                au_v[ki, sl] = bu + ki * 128
                ai_v[ki, sl] = bi + ki * 128
        ops = []
        for kb in range(4):
            for ki in range(8):
                k = kb * 8 + ki
                ops.append(pltpu.async_copy(
                    tabs[kb].at[au_v.at[ki]], eu_v.at[k], s))
                ops.append(pltpu.async_copy(
                    tabs[kb].at[ai_v.at[ki]], ei_v.at[k], s))
        return ops

    pend = {0: fire(0), 1: fire(1)}
    for cp in fcops:
        cp.wait()
    bias_s = bias_v[...][0]

    for c in range(_NCH):
        for cp in pend.pop(c):
            cp.wait()
        eu_v, ei_v = eu_slots[c % 2], ei_slots[c % 2]

        def body(g, carry, c=c, eu_v=eu_v, ei_v=ei_v):
            sl16 = pl.ds(c * _CH + g * _L, _L)
            gsl = pl.ds(g * _L, _L)
            ti = idxi_v[c, gsl]
            in_tail = ti >= _NMAIN
            toff = jnp.minimum(jnp.maximum(ti - _NMAIN, 0), _NTAIL - 1)
            acc = fu_v[sl16] + fi_v[sl16] + bias_s
            for k in range(_NF):
                au = eu_v[k, gsl]
                ai = eu_ai = ei_v[k, gsl]
                tv = plsc.load_gather(tail_v, [toff + k * _NTAIL])
                ai = jnp.where(in_tail, tv, eu_ai)
                acc = acc + au * ai
            out_v[sl16] = acc
            return carry

        lax.fori_loop(0, _NG, body, 0)
        if c + 2 < _NCH:
            pend[c + 2] = fire(c + 2)

    pltpu.sync_copy(out_v, out_hbm.at[pl.ds(base, _P)])


def kernel(users_feat, items_feat, emb_w, fc_w, bias):
    tabs = []
    for kb in range(4):
        t = emb_w[:_NMAIN, 8 * kb:8 * kb + 8]
        tabs.append(t.reshape(8593, 128, 8).transpose(0, 2, 1).reshape(-1))
    tail = emb_w[_NMAIN:, :].T.reshape(-1)
    return _fm_sc(users_feat, items_feat, *tabs, tail, fc_w.reshape(-1), bias)


# speedup vs baseline: 14.6708x; 1.0308x over previous
import functools

import jax
import jax.numpy as jnp
from jax import lax
from jax.experimental import pallas as pl
from jax.experimental.pallas import tpu as pltpu
from jax.experimental.pallas import tpu_sc as plsc

_N_USERS = 1000000
_N_ROWS = 1100000
_NMAIN = 1099904
_NTAIL = _N_ROWS - _NMAIN
_BATCH = 16384
_NF = 32
_L = 16

_info = plsc.get_sparse_core_info()
_NC, _NS = _info.num_cores, _info.num_subcores
_NW = _NC * _NS
_P = _BATCH // _NW
_CH = 128
_NCH = _P // _CH
_NG = _CH // _L

_mesh = plsc.VectorSubcoreMesh(core_axis_name="c", subcore_axis_name="s")


@functools.partial(
    pl.kernel,
    mesh=_mesh,
    out_type=jax.ShapeDtypeStruct((_BATCH,), jnp.float32),
    compiler_params=pltpu.CompilerParams(
        needs_layout_passes=False, use_tc_tiling_on_sc=False),
    scratch_types=[
        pltpu.VMEM((_NCH, _CH), jnp.int32),
        pltpu.VMEM((_NCH, _CH), jnp.int32),
        pltpu.VMEM((8, _CH), jnp.int32),
        pltpu.VMEM((8, _CH), jnp.int32),
        pltpu.VMEM((8, _CH), jnp.int32),
        pltpu.VMEM((8, _CH), jnp.int32),
        pltpu.VMEM((_NF, _CH), jnp.float32),
        pltpu.VMEM((_NF, _CH), jnp.float32),
        pltpu.VMEM((_NF, _CH), jnp.float32),
        pltpu.VMEM((_NF, _CH), jnp.float32),
        pltpu.VMEM((_P,), jnp.float32),
        pltpu.VMEM((_P,), jnp.float32),
        pltpu.VMEM((_P,), jnp.float32),
        pltpu.VMEM((_L,), jnp.float32),
        pltpu.VMEM((_NF * _NTAIL,), jnp.float32),
        pltpu.SemaphoreType.DMA,
        pltpu.SemaphoreType.DMA,
        pltpu.SemaphoreType.DMA,
    ],
)
def _fm_sc(users_hbm, items_hbm, t0_hbm, t1_hbm, t2_hbm, t3_hbm, tail_hbm,
           fc_hbm, bias_hbm, out_hbm,
           idxu_v, idxi_v, au0_v, au1_v, ai0_v, ai1_v,
           eu0_v, eu1_v, ei0_v, ei1_v,
           fu_v, fi_v, out_v, bias_v, tail_v, sem0, sem1, semf):
    wid = lax.axis_index("s") * _NC + lax.axis_index("c")
    base = wid * _P

    for c in range(_NCH):
        pltpu.sync_copy(users_hbm.at[pl.ds(base + c * _CH, _CH)], idxu_v.at[c])
        pltpu.sync_copy(items_hbm.at[pl.ds(base + c * _CH, _CH)], idxi_v.at[c])
    pltpu.sync_copy(bias_hbm, bias_v.at[pl.ds(0, 1)])
    pltpu.sync_copy(tail_hbm, tail_v)

    for c in range(_NCH):
        for j in range(_CH // _L):
            sl = pl.ds(j * _L, _L)
            idxi_v[c, sl] = idxi_v[c, sl] + _N_USERS

    fcops = []
    fc_row = fc_hbm.at[0]
    for c in range(_NCH):
        dst = pl.ds(c * _CH, _CH)
        fcops.append(pltpu.async_copy(fc_row.at[idxu_v.at[c]], fu_v.at[dst], semf))
        fcops.append(pltpu.async_copy(fc_row.at[idxi_v.at[c]], fi_v.at[dst], semf))

    tabs = (t0_hbm, t1_hbm, t2_hbm, t3_hbm)
    au_slots = (au0_v, au1_v)
    ai_slots = (ai0_v, ai1_v)
    eu_slots = (eu0_v, eu1_v)
    ei_slots = (ei0_v, ei1_v)

    def fire(c):
        s = sem0 if c % 2 == 0 else sem1
        slot = c % 2
        au_v, ai_v = au_slots[slot], ai_slots[slot]
        eu_v, ei_v = eu_slots[slot], ei_slots[slot]
        for j in range(_CH // _L):
            sl = pl.ds(j * _L, _L)
            tu = idxu_v[c, sl]
            bu = lax.shift_left(lax.shift_right_logical(tu, 7), 10) + (tu & 127)
            ti = jnp.minimum(idxi_v[c, sl], _NMAIN - 1)
            bi = lax.shift_left(lax.shift_right_logical(ti, 7), 10) + (ti & 127)
            for ki in range(8):
                au_v[ki, sl] = bu + ki * 128
                ai_v[ki, sl] = bi + ki * 128
        ops = []
        for kb in range(4):
            for ki in range(8):
                k = kb * 8 + ki
                ops.append(pltpu.async_copy(
                    tabs[kb].at[au_v.at[ki]], eu_v.at[k], s))
                ops.append(pltpu.async_copy(
                    tabs[kb].at[ai_v.at[ki]], ei_v.at[k], s))
        return ops

    pend = {0: fire(0), 1: fire(1)}
    for cp in fcops:
        cp.wait()
    bias_s = bias_v[...][0]

    for c in range(_NCH):
        for cp in pend.pop(c):
            cp.wait()
        eu_v, ei_v = eu_slots[c % 2], ei_slots[c % 2]

        def body(g, carry, c=c, eu_v=eu_v, ei_v=ei_v):
            sl16 = pl.ds(c * _CH + g * _L, _L)
            gsl = pl.ds(g * _L, _L)
            ti = idxi_v[c, gsl]
            in_tail = ti >= _NMAIN
            toff = jnp.minimum(jnp.maximum(ti - _NMAIN, 0), _NTAIL - 1)
            acc = fu_v[sl16] + fi_v[sl16] + bias_s
            for k in range(_NF):
                au = eu_v[k, gsl]
                ai = eu_ai = ei_v[k, gsl]
                tv = plsc.load_gather(tail_v, [toff + k * _NTAIL])
                ai = jnp.where(in_tail, tv, eu_ai)
                acc = acc + au * ai
            out_v[sl16] = acc
            return carry

        lax.fori_loop(0, _NG, body, 0)
        if c + 2 < _NCH:
            pend[c + 2] = fire(c + 2)

    pltpu.sync_copy(out_v, out_hbm.at[pl.ds(base, _P)])


def kernel(users_feat, items_feat, emb_w, fc_w, bias):
    tabs = []
    for kb in range(4):
        t = emb_w[:_NMAIN, 8 * kb:8 * kb + 8]
        tabs.append(t.reshape(8593, 128, 8).transpose(0, 2, 1).reshape(-1))
    tail = emb_w[_NMAIN:, :].T.reshape(-1)
    return _fm_sc(users_feat, items_feat, *tabs, tail, fc_w.T, bias)


# speedup vs baseline: 14.9340x; 1.0179x over previous
import functools

import jax
import jax.numpy as jnp
from jax import lax
from jax.experimental import pallas as pl
from jax.experimental.pallas import tpu as pltpu
from jax.experimental.pallas import tpu_sc as plsc

_N_USERS = 1000000
_N_ROWS = 1100000
_NMAIN = 1099904
_NTAIL = _N_ROWS - _NMAIN
_BATCH = 16384
_NF = 32
_L = 16

_info = plsc.get_sparse_core_info()
_NC, _NS = _info.num_cores, _info.num_subcores
_NW = _NC * _NS
_P = _BATCH // _NW
_CH = 128
_NCH = _P // _CH
_NG = _CH // _L

_mesh = plsc.VectorSubcoreMesh(core_axis_name="c", subcore_axis_name="s")


@functools.partial(
    pl.kernel,
    mesh=_mesh,
    out_type=jax.ShapeDtypeStruct((_BATCH,), jnp.float32),
    compiler_params=pltpu.CompilerParams(
        needs_layout_passes=False, use_tc_tiling_on_sc=False),
    scratch_types=[
        pltpu.VMEM((_NCH, _CH), jnp.int32),
        pltpu.VMEM((_NCH, _CH), jnp.int32),
        pltpu.VMEM((8, _CH), jnp.int32),
        pltpu.VMEM((8, _CH), jnp.int32),
        pltpu.VMEM((8, _CH), jnp.int32),
        pltpu.VMEM((8, _CH), jnp.int32),
        pltpu.VMEM((_NF, _CH), jnp.float32),
        pltpu.VMEM((_NF, _CH), jnp.float32),
        pltpu.VMEM((_NF, _CH), jnp.float32),
        pltpu.VMEM((_NF, _CH), jnp.float32),
        pltpu.VMEM((_P,), jnp.float32),
        pltpu.VMEM((_P,), jnp.float32),
        pltpu.VMEM((_P,), jnp.float32),
        pltpu.VMEM((_L,), jnp.float32),
        pltpu.VMEM((_NF * _NTAIL,), jnp.float32),
        pltpu.SemaphoreType.DMA,
        pltpu.SemaphoreType.DMA,
        pltpu.SemaphoreType.DMA,
    ],
)
def _fm_sc(users_hbm, items_hbm, t0_hbm, t1_hbm, t2_hbm, t3_hbm, tail_hbm,
           fc_hbm, bias_hbm, out_hbm,
           idxu_v, idxi_v, au0_v, au1_v, ai0_v, ai1_v,
           eu0_v, eu1_v, ei0_v, ei1_v,
           fu_v, fi_v, out_v, bias_v, tail_v, sem0, sem1, semf):
    wid = lax.axis_index("s") * _NC + lax.axis_index("c")
    base = wid * _P

    idxops = []
    for c in range(_NCH):
        idxops.append(pltpu.async_copy(
            users_hbm.at[pl.ds(base + c * _CH, _CH)], idxu_v.at[c], semf))
        idxops.append(pltpu.async_copy(
            items_hbm.at[pl.ds(base + c * _CH, _CH)], idxi_v.at[c], semf))
    idxops.append(pltpu.async_copy(bias_hbm, bias_v.at[pl.ds(0, 1)], semf))
    idxops.append(pltpu.async_copy(tail_hbm, tail_v, semf))
    for cp in idxops:
        cp.wait()

    for c in range(_NCH):
        for j in range(_CH // _L):
            sl = pl.ds(j * _L, _L)
            idxi_v[c, sl] = idxi_v[c, sl] + _N_USERS

    fcops = []
    fc_row = fc_hbm.at[0]
    for c in range(_NCH):
        dst = pl.ds(c * _CH, _CH)
        fcops.append(pltpu.async_copy(fc_row.at[idxu_v.at[c]], fu_v.at[dst], semf))
        fcops.append(pltpu.async_copy(fc_row.at[idxi_v.at[c]], fi_v.at[dst], semf))

    tabs = (t0_hbm, t1_hbm, t2_hbm, t3_hbm)
    au_slots = (au0_v, au1_v)
    ai_slots = (ai0_v, ai1_v)
    eu_slots = (eu0_v, eu1_v)
    ei_slots = (ei0_v, ei1_v)

    def fire(c):
        s = sem0 if c % 2 == 0 else sem1
        slot = c % 2
        au_v, ai_v = au_slots[slot], ai_slots[slot]
        eu_v, ei_v = eu_slots[slot], ei_slots[slot]
        for j in range(_CH // _L):
            sl = pl.ds(j * _L, _L)
            tu = idxu_v[c, sl]
            bu = lax.shift_left(lax.shift_right_logical(tu, 7), 10) + (tu & 127)
            ti = jnp.minimum(idxi_v[c, sl], _NMAIN - 1)
            bi = lax.shift_left(lax.shift_right_logical(ti, 7), 10) + (ti & 127)
            for ki in range(8):
                au_v[ki, sl] = bu + ki * 128
                ai_v[ki, sl] = bi + ki * 128
        ops = []
        for kb in range(4):
            for ki in range(8):
                k = kb * 8 + ki
                ops.append(pltpu.async_copy(
                    tabs[kb].at[au_v.at[ki]], eu_v.at[k], s))
                ops.append(pltpu.async_copy(
                    tabs[kb].at[ai_v.at[ki]], ei_v.at[k], s))
        return ops

    pend = {0: fire(0), 1: fire(1)}
    for cp in fcops:
        cp.wait()
    bias_s = bias_v[...][0]

    for c in range(_NCH):
        for cp in pend.pop(c):
            cp.wait()
        eu_v, ei_v = eu_slots[c % 2], ei_slots[c % 2]

        def body(g, carry, c=c, eu_v=eu_v, ei_v=ei_v):
            sl16 = pl.ds(c * _CH + g * _L, _L)
            gsl = pl.ds(g * _L, _L)
            ti = idxi_v[c, gsl]
            in_tail = ti >= _NMAIN
            toff = jnp.minimum(jnp.maximum(ti - _NMAIN, 0), _NTAIL - 1)
            acc = fu_v[sl16] + fi_v[sl16] + bias_s
            for k in range(_NF):
                au = eu_v[k, gsl]
                ai = eu_ai = ei_v[k, gsl]
                tv = plsc.load_gather(tail_v, [toff + k * _NTAIL])
                ai = jnp.where(in_tail, tv, eu_ai)
                acc = acc + au * ai
            out_v[sl16] = acc
            return carry

        lax.fori_loop(0, _NG, body, 0)
        if c + 2 < _NCH:
            pend[c + 2] = fire(c + 2)

    pltpu.sync_copy(out_v, out_hbm.at[pl.ds(base, _P)])


def kernel(users_feat, items_feat, emb_w, fc_w, bias):
    tabs = []
    for kb in range(4):
        t = emb_w[:_NMAIN, 8 * kb:8 * kb + 8]
        tabs.append(t.reshape(8593, 128, 8).transpose(0, 2, 1).reshape(-1))
    tail = emb_w[_NMAIN:, :].T.reshape(-1)
    return _fm_sc(users_feat, items_feat, *tabs, tail, fc_w.T, bias)
